# Initial kernel scaffold; baseline (speedup 1.0000x reference)
import functools

import jax
import jax.numpy as jnp
from jax import lax
from jax.experimental import pallas as pl

NUM_GRAPHS = 512
IN_DIM = 256
N_NODES = 50000
BLK = 1000
NBLK = N_NODES // BLK


def _body(node_ref, ids_ref, w1_ref, b1_ref, w2_ref, b2_ref,
          g_ref, z_ref, acc_ref, cnt_ref):
    j = pl.program_id(0)
    x = node_ref[...]
    ids = ids_ref[0, 0, :]
    iota = lax.broadcasted_iota(jnp.int32, (BLK, NUM_GRAPHS), 1)
    oh = (ids[:, None] == iota).astype(jnp.float32)
    partial = lax.dot_general(oh, x, (((0,), (0,)), ((), ())),
                              preferred_element_type=jnp.float32)
    c = jnp.sum(oh, axis=0)[None, :]

    @pl.when(j == 0)
    def _init():
        acc_ref[...] = partial
        cnt_ref[...] = c

    @pl.when(j > 0)
    def _accum():
        acc_ref[...] += partial
        cnt_ref[...] += c

    @pl.when(j == NBLK - 1)
    def _finalize():
        counts = jnp.maximum(cnt_ref[0, :], 1.0)
        g = acc_ref[...] / counts[:, None]
        g_ref[...] = g
        h = lax.dot_general(g, w1_ref[...], (((1,), (1,)), ((), ())),
                            preferred_element_type=jnp.float32) + b1_ref[0, :]
        h = jnp.maximum(h, 0.0)
        z_ref[...] = lax.dot_general(h, w2_ref[...], (((1,), (1,)), ((), ())),
                                     preferred_element_type=jnp.float32) + b2_ref[0, :]


@jax.jit
def kernel(node_rep, batch_ids, W1, b1, W2, b2):
    ids = batch_ids.astype(jnp.int32).reshape(NBLK, 1, BLK)
    grid = (NBLK,)
    out = pl.pallas_call(
        _body,
        grid=grid,
        in_specs=[
            pl.BlockSpec((BLK, IN_DIM), lambda j: (j, 0)),
            pl.BlockSpec((1, 1, BLK), lambda j: (j, 0, 0)),
            pl.BlockSpec((IN_DIM, IN_DIM), lambda j: (0, 0)),
            pl.BlockSpec((1, IN_DIM), lambda j: (0, 0)),
            pl.BlockSpec((IN_DIM, IN_DIM), lambda j: (0, 0)),
            pl.BlockSpec((1, IN_DIM), lambda j: (0, 0)),
        ],
        out_specs=[
            pl.BlockSpec((NUM_GRAPHS, IN_DIM), lambda j: (0, 0)),
            pl.BlockSpec((NUM_GRAPHS, IN_DIM), lambda j: (0, 0)),
        ],
        out_shape=[
            jax.ShapeDtypeStruct((NUM_GRAPHS, IN_DIM), jnp.float32),
            jax.ShapeDtypeStruct((NUM_GRAPHS, IN_DIM), jnp.float32),
        ],
        scratch_shapes=[
            pltpu.VMEM((NUM_GRAPHS, IN_DIM), jnp.float32),
            pltpu.VMEM((1, NUM_GRAPHS), jnp.float32),
        ],
    )(node_rep, ids, W1, b1.reshape(1, IN_DIM), W2, b2.reshape(1, IN_DIM))
    return (out[0], out[1])


from jax.experimental.pallas import tpu as pltpu


# speedup vs baseline: 8.8381x; 8.8381x over previous
import functools

import jax
import jax.numpy as jnp
from jax import lax
from jax.experimental import pallas as pl
from jax.experimental.pallas import tpu as pltpu

NUM_GRAPHS = 512
IN_DIM = 256
N_NODES = 50000
BLK = 1000
NBLK = N_NODES // BLK


def _body(node_ref, ids_ref, w1_ref, b1_ref, w2_ref, b2_ref,
          g_ref, z_ref, acc_ref, cnt_ref):
    j = pl.program_id(0)
    x = node_ref[...]
    ids = ids_ref[0, 0, :]
    iota = lax.broadcasted_iota(jnp.int32, (BLK, NUM_GRAPHS), 1)
    oh = (ids[:, None] == iota).astype(jnp.float32)
    partial = lax.dot_general(oh, x, (((0,), (0,)), ((), ())),
                              preferred_element_type=jnp.float32)
    c = jnp.sum(oh, axis=0)[None, :]

    @pl.when(j == 0)
    def _init():
        acc_ref[...] = partial
        cnt_ref[...] = c

    @pl.when(j > 0)
    def _accum():
        acc_ref[...] += partial
        cnt_ref[...] += c

    @pl.when(j == NBLK - 1)
    def _finalize():
        counts = jnp.maximum(cnt_ref[0, :], 1.0)
        g = acc_ref[...] / counts[:, None]
        g_ref[...] = g
        h = lax.dot_general(g, w1_ref[...], (((1,), (1,)), ((), ())),
                            preferred_element_type=jnp.float32) + b1_ref[0, :]
        h = jnp.maximum(h, 0.0)
        z_ref[...] = lax.dot_general(h, w2_ref[...], (((1,), (1,)), ((), ())),
                                     preferred_element_type=jnp.float32) + b2_ref[0, :]


@jax.jit
def kernel(node_rep, batch_ids, W1, b1, W2, b2):
    ids = batch_ids.astype(jnp.int32).reshape(NBLK, 1, BLK)
    grid = (NBLK,)
    out = pl.pallas_call(
        _body,
        grid=grid,
        in_specs=[
            pl.BlockSpec((BLK, IN_DIM), lambda j: (j, 0)),
            pl.BlockSpec((1, 1, BLK), lambda j: (j, 0, 0)),
            pl.BlockSpec((IN_DIM, IN_DIM), lambda j: (0, 0)),
            pl.BlockSpec((1, IN_DIM), lambda j: (0, 0)),
            pl.BlockSpec((IN_DIM, IN_DIM), lambda j: (0, 0)),
            pl.BlockSpec((1, IN_DIM), lambda j: (0, 0)),
        ],
        out_specs=[
            pl.BlockSpec((NUM_GRAPHS, IN_DIM), lambda j: (0, 0)),
            pl.BlockSpec((NUM_GRAPHS, IN_DIM), lambda j: (0, 0)),
        ],
        out_shape=[
            jax.ShapeDtypeStruct((NUM_GRAPHS, IN_DIM), jnp.float32),
            jax.ShapeDtypeStruct((NUM_GRAPHS, IN_DIM), jnp.float32),
        ],
        scratch_shapes=[
            pltpu.VMEM((NUM_GRAPHS, IN_DIM), jnp.float32),
            pltpu.VMEM((1, NUM_GRAPHS), jnp.float32),
        ],
    )(node_rep, ids, W1, b1.reshape(1, IN_DIM), W2, b2.reshape(1, IN_DIM))
    return (out[0], out[1])
